# baseline (speedup 1.0000x reference)
import jax
import jax.numpy as jnp
from jax.experimental import pallas as pl
from jax.experimental.pallas import tpu as pltpu

HIDDEN = 1024
HEADS = 16
ATT = HIDDEN // HEADS
SCALE = ATT ** -0.5


def _u_kernel(q_ref, wq_ref, bq_ref, wk_ref, u_ref):
    qh = jax.lax.dot_general(q_ref[...], wq_ref[...], (((1,), (0,)), ((), ())),
                             preferred_element_type=jnp.float32)
    qh = (qh + bq_ref[...]) * SCALE
    for h in range(HEADS):
        qs = qh[:, h * ATT:(h + 1) * ATT]
        ws = wk_ref[:, h * ATT:(h + 1) * ATT]
        u_ref[:, h, :] = jax.lax.dot_general(
            qs, ws, (((1,), (1,)), ((), ())),
            preferred_element_type=jnp.float32)


def _mha_kernel(u_ref, k_ref, v_ref, wv_ref, bv_ref, wo_ref, bo_ref, out_ref):
    u = u_ref[0]
    k = k_ref[0]
    scores = jax.lax.dot_general(k, u, (((1,), (1,)), ((), ())),
                                 preferred_element_type=jnp.float32)
    m = jnp.max(scores, axis=0, keepdims=True)
    e = jnp.exp(scores - m)
    attn = e * (1.0 / jnp.sum(e, axis=0, keepdims=True))
    vv = v_ref[0]
    a = jax.lax.dot_general(attn, vv, (((0,), (0,)), ((), ())),
                            preferred_element_type=jnp.float32)
    f = jax.lax.dot_general(a, wv_ref[...], (((1,), (0,)), ((), ())),
                            preferred_element_type=jnp.float32)
    col_head = jax.lax.broadcasted_iota(jnp.int32, (HEADS, HIDDEN), 1) // ATT
    row_head = jax.lax.broadcasted_iota(jnp.int32, (HEADS, HIDDEN), 0)
    mask = (col_head == row_head).astype(jnp.float32)
    upd = jnp.sum(f * mask, axis=0, keepdims=True) + bv_ref[...]
    out_ref[0] = jax.lax.dot_general(
        upd, wo_ref[...], (((1,), (0,)), ((), ())),
        preferred_element_type=jnp.float32) + bo_ref[...]


def kernel(q, k, v, Wq, bq, Wk, bk, Wv, bv, Wo, bo):
    del bk
    B = q.shape[0]
    L_K = k.shape[1]
    full = lambda b: (0, 0)
    u = pl.pallas_call(
        _u_kernel,
        in_specs=[
            pl.BlockSpec((B, HIDDEN), lambda: (0, 0)),
            pl.BlockSpec((HIDDEN, HIDDEN), lambda: (0, 0)),
            pl.BlockSpec((1, HIDDEN), lambda: (0, 0)),
            pl.BlockSpec((HIDDEN, HIDDEN), lambda: (0, 0)),
        ],
        out_specs=pl.BlockSpec((B, HEADS, HIDDEN), lambda: (0, 0, 0)),
        out_shape=jax.ShapeDtypeStruct((B, HEADS, HIDDEN), jnp.float32),
    )(q, Wq, bq.reshape(1, HIDDEN), Wk)
    out = pl.pallas_call(
        _mha_kernel,
        grid=(B,),
        in_specs=[
            pl.BlockSpec((1, HEADS, HIDDEN), lambda b: (b, 0, 0)),
            pl.BlockSpec((1, L_K, HIDDEN), lambda b: (b, 0, 0)),
            pl.BlockSpec((1, L_K, HIDDEN), lambda b: (b, 0, 0)),
            pl.BlockSpec((HIDDEN, HIDDEN), full),
            pl.BlockSpec((1, HIDDEN), full),
            pl.BlockSpec((HIDDEN, HIDDEN), full),
            pl.BlockSpec((1, HIDDEN), full),
        ],
        out_specs=pl.BlockSpec((1, 1, HIDDEN), lambda b: (b, 0, 0)),
        out_shape=jax.ShapeDtypeStruct((B, 1, HIDDEN), jnp.float32),
        compiler_params=pltpu.CompilerParams(
            dimension_semantics=("parallel",)),
    )(u, k, v, Wv, bv.reshape(1, HIDDEN), Wo, bo.reshape(1, HIDDEN))
    return out.reshape(B, HIDDEN)
